# baseline (speedup 1.0000x reference)
import functools

import jax
import jax.numpy as jnp
from jax import lax
from jax.experimental import pallas as pl
from jax.experimental.pallas import tpu as pltpu
from jax.experimental.pallas import tpu_sc as plsc

N = 100000
MAX_DEG_PLUS1 = 513
D_FEAT = 128
D_NODE = 128
D_ENC = 64
OUT_W = 1 + D_FEAT + D_NODE + D_ENC

NW = 32
N_SUB = 16
CHUNK = 400
A_ROWS = 5200
B_ROWS = 1200
PAIR_ROWS = A_ROWS + B_ROWS
B_PAD = N_SUB * PAIR_ROWS

R = 4096


def _sc_gather(degree_pad, table):
    mesh = plsc.VectorSubcoreMesh(core_axis_name="c", subcore_axis_name="s")

    @functools.partial(
        pl.kernel,
        mesh=mesh,
        out_type=jax.ShapeDtypeStruct((B_PAD, D_ENC), jnp.float32),
        scratch_types=[
            pltpu.VMEM((A_ROWS,), jnp.int32),
            pltpu.VMEM((CHUNK, D_ENC), jnp.float32),
            pltpu.VMEM((CHUNK, D_ENC), jnp.float32),
            pltpu.SemaphoreType.DMA,
            pltpu.SemaphoreType.DMA,
            pltpu.SemaphoreType.DMA,
            pltpu.SemaphoreType.DMA,
        ],
        compiler_params=pltpu.CompilerParams(use_tc_tiling_on_sc=False),
    )
    def k(deg_hbm, table_hbm, out_hbm, idx_v, rows_a, rows_b, ga, gb, wa, wb):
        c = lax.axis_index("c")
        s = lax.axis_index("s")
        base = s * PAIR_ROWS + c * A_ROWS
        bufs = (rows_a, rows_b)
        gsems = (ga, gb)
        wsems = (wa, wb)

        def run(nrows):
            nchunks = nrows // CHUNK
            pltpu.sync_copy(
                deg_hbm.at[pl.ds(base, nrows)], idx_v.at[pl.ds(0, nrows)]
            )

            def gather(ci, buf, sem):
                return pltpu.async_copy(
                    table_hbm.at[idx_v.at[pl.ds(ci * CHUNK, CHUNK)]], buf, sem
                )

            g = [gather(0, bufs[0], gsems[0]), None]
            w = [None, None]
            for ci in range(nchunks):
                b = ci % 2
                nb = 1 - b
                if ci + 1 < nchunks:
                    if w[nb] is not None:
                        w[nb].wait()
                        w[nb] = None
                    g[nb] = gather(ci + 1, bufs[nb], gsems[nb])
                g[b].wait()
                w[b] = pltpu.async_copy(
                    bufs[b], out_hbm.at[pl.ds(base + ci * CHUNK, CHUNK)], wsems[b]
                )
            for b in range(2):
                if w[b] is not None:
                    w[b].wait()

        @pl.when(c == 0)
        def _():
            run(A_ROWS)

        @pl.when(c == 1)
        def _():
            run(B_ROWS)

    return k(degree_pad, table)


def _assemble_t(feats, node_w, deg_pairs):
    def body(f_ref, n_ref, d_ref, o_ref):
        z = jnp.zeros((1, R), jnp.float32)
        f_t = jnp.transpose(f_ref[...], (1, 0))
        n_t = jnp.transpose(n_ref[...], (1, 0))
        d_t_full = jnp.transpose(d_ref[...], (1, 0))
        a = d_t_full[:D_ENC]
        b = d_t_full[D_ENC:]
        rowi = lax.broadcasted_iota(jnp.int32, (R // 2, R), 0)
        coli = lax.broadcasted_iota(jnp.int32, (R // 2, R), 1)
        pa = (coli == 2 * rowi).astype(jnp.float32)
        pb = (coli == 2 * rowi + 1).astype(jnp.float32)
        d_t = lax.dot(a, pa, preferred_element_type=jnp.float32) + lax.dot(
            b, pb, preferred_element_type=jnp.float32
        )
        o_ref[...] = jnp.concatenate([z, f_t, n_t, d_t], axis=0)

    return pl.pallas_call(
        body,
        grid=(pl.cdiv(N, R),),
        in_specs=[
            pl.BlockSpec((R, D_FEAT), lambda i: (i, 0)),
            pl.BlockSpec((R, D_NODE), lambda i: (i, 0)),
            pl.BlockSpec((R // 2, 128), lambda i: (i, 0)),
        ],
        out_specs=pl.BlockSpec((OUT_W, R), lambda i: (0, i)),
        out_shape=jax.ShapeDtypeStruct((OUT_W, N), jnp.float32),
    )(feats, node_w, deg_pairs)


def kernel(feats, degree, edge_batch, emb_node_w, emb_degree_w):
    del edge_batch
    deg_pad = jnp.concatenate(
        [degree.astype(jnp.int32), jnp.zeros((B_PAD - N,), jnp.int32)]
    )
    REP = 8
    table_rep = jnp.tile(emb_degree_w, (REP, 1))
    r = jnp.arange(B_PAD, dtype=jnp.int32)
    s = r // PAIR_ROWS
    cc = ((r % PAIR_ROWS) >= A_ROWS).astype(jnp.int32)
    offs = ((s * 2 + cc) % REP) * MAX_DEG_PLUS1
    deg_emb = _sc_gather(deg_pad + offs, table_rep)
    deg_pairs = jnp.reshape(deg_emb, (B_PAD // 2, 128))
    return jnp.transpose(_assemble_t(feats, emb_node_w, deg_pairs), (1, 0))


# speedup vs baseline: 1.2146x; 1.2146x over previous
import functools

import jax
import jax.numpy as jnp
from jax import lax
from jax.experimental import pallas as pl
from jax.experimental.pallas import tpu as pltpu
from jax.experimental.pallas import tpu_sc as plsc

N = 100000
MAX_DEG_PLUS1 = 513
D_FEAT = 128
D_NODE = 128
D_ENC = 64
OUT_W = 1 + D_FEAT + D_NODE + D_ENC

NW = 32
N_SUB = 16
CHUNK = 400
A_ROWS = 5200
B_ROWS = 1200
PAIR_ROWS = A_ROWS + B_ROWS
B_PAD = N_SUB * PAIR_ROWS

R = 1536


def _sc_gather(degree_pad, table):
    mesh = plsc.VectorSubcoreMesh(core_axis_name="c", subcore_axis_name="s")

    @functools.partial(
        pl.kernel,
        mesh=mesh,
        out_type=jax.ShapeDtypeStruct((B_PAD, D_ENC), jnp.float32),
        scratch_types=[
            pltpu.VMEM((A_ROWS,), jnp.int32),
            pltpu.VMEM((CHUNK, D_ENC), jnp.float32),
            pltpu.VMEM((CHUNK, D_ENC), jnp.float32),
            pltpu.SemaphoreType.DMA,
            pltpu.SemaphoreType.DMA,
            pltpu.SemaphoreType.DMA,
            pltpu.SemaphoreType.DMA,
        ],
        compiler_params=pltpu.CompilerParams(use_tc_tiling_on_sc=False),
    )
    def k(deg_hbm, table_hbm, out_hbm, idx_v, rows_a, rows_b, ga, gb, wa, wb):
        c = lax.axis_index("c")
        s = lax.axis_index("s")
        base = s * PAIR_ROWS + c * A_ROWS
        bufs = (rows_a, rows_b)
        gsems = (ga, gb)
        wsems = (wa, wb)

        def run(nrows):
            nchunks = nrows // CHUNK
            pltpu.sync_copy(
                deg_hbm.at[pl.ds(base, nrows)], idx_v.at[pl.ds(0, nrows)]
            )

            def gather(ci, buf, sem):
                return pltpu.async_copy(
                    table_hbm.at[idx_v.at[pl.ds(ci * CHUNK, CHUNK)]], buf, sem
                )

            g = [gather(0, bufs[0], gsems[0]), None]
            w = [None, None]
            for ci in range(nchunks):
                b = ci % 2
                nb = 1 - b
                if ci + 1 < nchunks:
                    if w[nb] is not None:
                        w[nb].wait()
                        w[nb] = None
                    g[nb] = gather(ci + 1, bufs[nb], gsems[nb])
                g[b].wait()
                w[b] = pltpu.async_copy(
                    bufs[b], out_hbm.at[pl.ds(base + ci * CHUNK, CHUNK)], wsems[b]
                )
            for b in range(2):
                if w[b] is not None:
                    w[b].wait()

        @pl.when(c == 0)
        def _():
            run(A_ROWS)

        @pl.when(c == 1)
        def _():
            run(B_ROWS)

    return k(degree_pad, table)


def _assemble_t(feats, node_w, deg_pairs):
    def body(f_ref, n_ref, d_ref, o_ref):
        z = jnp.zeros((1, R), jnp.float32)
        f_t = jnp.transpose(f_ref[...], (1, 0))
        n_t = jnp.transpose(n_ref[...], (1, 0))
        d_t_full = jnp.transpose(d_ref[...], (1, 0))
        a = d_t_full[:D_ENC]
        b = d_t_full[D_ENC:]
        rowi = lax.broadcasted_iota(jnp.int32, (R // 2, R), 0)
        coli = lax.broadcasted_iota(jnp.int32, (R // 2, R), 1)
        pa = (coli == 2 * rowi).astype(jnp.float32)
        pb = (coli == 2 * rowi + 1).astype(jnp.float32)
        d_t = lax.dot(a, pa, preferred_element_type=jnp.float32) + lax.dot(
            b, pb, preferred_element_type=jnp.float32
        )
        o_ref[...] = jnp.concatenate([z, f_t, n_t, d_t], axis=0)

    return pl.pallas_call(
        body,
        grid=(pl.cdiv(N, R),),
        in_specs=[
            pl.BlockSpec((R, D_FEAT), lambda i: (i, 0)),
            pl.BlockSpec((R, D_NODE), lambda i: (i, 0)),
            pl.BlockSpec((R // 2, 128), lambda i: (i, 0)),
        ],
        out_specs=pl.BlockSpec((OUT_W, R), lambda i: (0, i)),
        out_shape=jax.ShapeDtypeStruct((OUT_W, N), jnp.float32),
    )(feats, node_w, deg_pairs)


def kernel(feats, degree, edge_batch, emb_node_w, emb_degree_w):
    del edge_batch
    deg_pad = jnp.concatenate(
        [degree.astype(jnp.int32), jnp.zeros((B_PAD - N,), jnp.int32)]
    )
    REP = 8
    table_rep = jnp.tile(emb_degree_w, (REP, 1))
    r = jnp.arange(B_PAD, dtype=jnp.int32)
    s = r // PAIR_ROWS
    cc = ((r % PAIR_ROWS) >= A_ROWS).astype(jnp.int32)
    offs = ((s * 2 + cc) % REP) * MAX_DEG_PLUS1
    deg_emb = _sc_gather(deg_pad + offs, table_rep)
    deg_pairs = jnp.reshape(deg_emb, (B_PAD // 2, 128))
    return jnp.transpose(_assemble_t(feats, emb_node_w, deg_pairs), (1, 0))


# speedup vs baseline: 1.2376x; 1.0189x over previous
import functools

import jax
import jax.numpy as jnp
from jax import lax
from jax.experimental import pallas as pl
from jax.experimental.pallas import tpu as pltpu
from jax.experimental.pallas import tpu_sc as plsc

N = 100000
MAX_DEG_PLUS1 = 513
D_FEAT = 128
D_NODE = 128
D_ENC = 64
OUT_W = 1 + D_FEAT + D_NODE + D_ENC

NW = 32
N_SUB = 16
CHUNK = 400
A_ROWS = 5200
B_ROWS = 1200
PAIR_ROWS = A_ROWS + B_ROWS
B_PAD = N_SUB * PAIR_ROWS

R = 2048


def _sc_gather(degree_pad, table):
    mesh = plsc.VectorSubcoreMesh(core_axis_name="c", subcore_axis_name="s")

    @functools.partial(
        pl.kernel,
        mesh=mesh,
        out_type=jax.ShapeDtypeStruct((B_PAD, D_ENC), jnp.float32),
        scratch_types=[
            pltpu.VMEM((A_ROWS,), jnp.int32),
            pltpu.VMEM((CHUNK, D_ENC), jnp.float32),
            pltpu.VMEM((CHUNK, D_ENC), jnp.float32),
            pltpu.SemaphoreType.DMA,
            pltpu.SemaphoreType.DMA,
            pltpu.SemaphoreType.DMA,
            pltpu.SemaphoreType.DMA,
        ],
        compiler_params=pltpu.CompilerParams(use_tc_tiling_on_sc=False),
    )
    def k(deg_hbm, table_hbm, out_hbm, idx_v, rows_a, rows_b, ga, gb, wa, wb):
        c = lax.axis_index("c")
        s = lax.axis_index("s")
        base = s * PAIR_ROWS + c * A_ROWS
        bufs = (rows_a, rows_b)
        gsems = (ga, gb)
        wsems = (wa, wb)

        def run(nrows):
            nchunks = nrows // CHUNK
            pltpu.sync_copy(
                deg_hbm.at[pl.ds(base, nrows)], idx_v.at[pl.ds(0, nrows)]
            )

            def gather(ci, buf, sem):
                return pltpu.async_copy(
                    table_hbm.at[idx_v.at[pl.ds(ci * CHUNK, CHUNK)]], buf, sem
                )

            g = [gather(0, bufs[0], gsems[0]), None]
            w = [None, None]
            for ci in range(nchunks):
                b = ci % 2
                nb = 1 - b
                if ci + 1 < nchunks:
                    if w[nb] is not None:
                        w[nb].wait()
                        w[nb] = None
                    g[nb] = gather(ci + 1, bufs[nb], gsems[nb])
                g[b].wait()
                w[b] = pltpu.async_copy(
                    bufs[b], out_hbm.at[pl.ds(base + ci * CHUNK, CHUNK)], wsems[b]
                )
            for b in range(2):
                if w[b] is not None:
                    w[b].wait()

        @pl.when(c == 0)
        def _():
            run(A_ROWS)

        @pl.when(c == 1)
        def _():
            run(B_ROWS)

    return k(degree_pad, table)


def _assemble_t(feats, node_w, deg_pairs):
    def body(f_ref, n_ref, d_ref, o_ref):
        z = jnp.zeros((1, R), jnp.float32)
        f_t = jnp.transpose(f_ref[...], (1, 0))
        n_t = jnp.transpose(n_ref[...], (1, 0))
        d_t_full = jnp.transpose(d_ref[...], (1, 0))
        a = d_t_full[:D_ENC]
        b = d_t_full[D_ENC:]
        rowi = lax.broadcasted_iota(jnp.int32, (R // 2, R), 0)
        coli = lax.broadcasted_iota(jnp.int32, (R // 2, R), 1)
        pa = (coli == 2 * rowi).astype(jnp.float32)
        pb = (coli == 2 * rowi + 1).astype(jnp.float32)
        d_t = lax.dot(a, pa, preferred_element_type=jnp.float32) + lax.dot(
            b, pb, preferred_element_type=jnp.float32
        )
        o_ref[...] = jnp.concatenate([z, f_t, n_t, d_t], axis=0)

    return pl.pallas_call(
        body,
        grid=(pl.cdiv(N, R),),
        in_specs=[
            pl.BlockSpec((R, D_FEAT), lambda i: (i, 0)),
            pl.BlockSpec((R, D_NODE), lambda i: (i, 0)),
            pl.BlockSpec((R // 2, 128), lambda i: (i, 0)),
        ],
        out_specs=pl.BlockSpec((OUT_W, R), lambda i: (0, i)),
        out_shape=jax.ShapeDtypeStruct((OUT_W, N), jnp.float32),
    )(feats, node_w, deg_pairs)


def kernel(feats, degree, edge_batch, emb_node_w, emb_degree_w):
    del edge_batch
    deg_pad = jnp.concatenate(
        [degree.astype(jnp.int32), jnp.zeros((B_PAD - N,), jnp.int32)]
    )
    REP = 4
    table_rep = jnp.tile(emb_degree_w, (REP, 1))
    r = jnp.arange(B_PAD, dtype=jnp.int32)
    s = r // PAIR_ROWS
    cc = ((r % PAIR_ROWS) >= A_ROWS).astype(jnp.int32)
    offs = ((s * 2 + cc) % REP) * MAX_DEG_PLUS1
    deg_emb = _sc_gather(deg_pad + offs, table_rep)
    deg_pairs = jnp.reshape(deg_emb, (B_PAD // 2, 128))
    return jnp.transpose(_assemble_t(feats, emb_node_w, deg_pairs), (1, 0))
